# Initial kernel scaffold; baseline (speedup 1.0000x reference)
import jax
import jax.numpy as jnp
from jax.experimental import pallas as pl


def kernel(x, edge_attr, params, edge_index, batch):
    raise NotImplementedError("write your pallas kernel here")


# speedup vs baseline: 3.1904x; 3.1904x over previous
import functools
import math

import jax
import jax.numpy as jnp
from jax import lax
from jax.experimental import pallas as pl
from jax.experimental.pallas import tpu as pltpu
from jax.experimental.pallas import tpu_sc as plsc

N = 4096
E = 65536
DIN = 128
DE = 16
G = 32
NB = 8
BN = N // NB
EB = 64
BE = E // EB


def _ln(x, g, b):
    m = x.mean(-1, keepdims=True)
    v = ((x - m) ** 2).mean(-1, keepdims=True)
    return (x - m) * jax.lax.rsqrt(v + 1e-5) * g + b


def _pre_body(x_ref, inW_ref, inb_ref, lng_ref, lnb_ref, qkvW_ref, qkvb_ref,
              x1_ref, qkv_ref):
    x = x_ref[...]
    x1 = _ln(x @ inW_ref[...] + inb_ref[...], lng_ref[...], lnb_ref[...])
    x1_ref[...] = x1
    qkv_ref[...] = x1 @ qkvW_ref[...].T + qkvb_ref[...]


def _pre(x, tr):
    full = lambda s: pl.BlockSpec(s, lambda i: (0,) * len(s))
    return pl.pallas_call(
        _pre_body,
        grid=(NB,),
        in_specs=[pl.BlockSpec((BN, DIN), lambda i: (i, 0)),
                  full((DIN, 128)), full((1, 128)), full((1, 128)),
                  full((1, 128)), full((384, 128)), full((1, 384))],
        out_specs=[pl.BlockSpec((BN, 128), lambda i: (i, 0)),
                   pl.BlockSpec((BN, 384), lambda i: (i, 0))],
        out_shape=[jax.ShapeDtypeStruct((N, 128), jnp.float32),
                   jax.ShapeDtypeStruct((N, 384), jnp.float32)],
    )(x, tr['in_W'], tr['in_b'], tr['ln_g'], tr['ln_b'], tr['qkv_W'],
      tr['qkv_b'])


def _attn_body(q_ref, kv_ref, o_ref):
    qblk = q_ref[...]
    kv = kv_ref[...]
    outs = []
    for h in range(4):
        q = qblk[:, 32 * h:32 * h + 32]
        k = kv[:, 128 + 32 * h:128 + 32 * h + 32]
        v = kv[:, 256 + 32 * h:256 + 32 * h + 32]
        s = (q @ k.T) * (1.0 / math.sqrt(32.0))
        m = s.max(-1, keepdims=True)
        p = jnp.exp(s - m)
        p = p / p.sum(-1, keepdims=True)
        outs.append(p @ v)
    o_ref[...] = jnp.concatenate(outs, axis=-1)


def _attn(qkv):
    return pl.pallas_call(
        _attn_body,
        grid=(NB,),
        in_specs=[pl.BlockSpec((BN, 384), lambda i: (i, 0)),
                  pl.BlockSpec((N, 384), lambda i: (0, 0))],
        out_specs=pl.BlockSpec((BN, 128), lambda i: (i, 0)),
        out_shape=jax.ShapeDtypeStruct((N, 128), jnp.float32),
    )(qkv, qkv)


def _post_body(x1_ref, o_ref, outW_ref, outb_ref, lng_ref, lnb_ref,
               ff1W_ref, ff1b_ref, ff2W_ref, ff2b_ref, oW_ref, ob_ref,
               lnog_ref, lnob_ref, Wl_ref, bl_ref, Wr_ref, br_ref, *outs):
    x = x1_ref[...] + (o_ref[...] @ outW_ref[...].T + outb_ref[...])
    x = _ln(x, lng_ref[...], lnb_ref[...])
    ff = jnp.maximum(x @ ff1W_ref[...] + ff1b_ref[...], 0.0) @ ff2W_ref[...] \
        + ff2b_ref[...]
    x = _ln(x + ff, lng_ref[...], lnb_ref[...])
    x = x @ oW_ref[...] + ob_ref[...]
    x = _ln(x, lnog_ref[...], lnob_ref[...])
    xl = x @ Wl_ref[...] + bl_ref[...]
    xr = x @ Wr_ref[...] + br_ref[...]
    for h in range(4):
        outs[h][...] = xl[:, h * 256:(h + 1) * 256]
        outs[4 + h][...] = xr[:, h * 256:(h + 1) * 256]


def _post(x1, o, tr, c1):
    full = lambda s: pl.BlockSpec(s, lambda i: (0,) * len(s))
    return pl.pallas_call(
        _post_body,
        grid=(NB,),
        in_specs=[pl.BlockSpec((BN, 128), lambda i: (i, 0)),
                  pl.BlockSpec((BN, 128), lambda i: (i, 0)),
                  full((128, 128)), full((1, 128)), full((1, 128)),
                  full((1, 128)), full((128, 512)), full((1, 512)),
                  full((512, 128)), full((1, 128)), full((128, 64)),
                  full((1, 64)), full((1, 64)), full((1, 64)),
                  full((64, 1024)), full((1, 1024)),
                  full((64, 1024)), full((1, 1024))],
        out_specs=[pl.BlockSpec((BN, 256), lambda i: (i, 0))] * 8,
        out_shape=[jax.ShapeDtypeStruct((N, 256), jnp.float32)] * 8,
    )(x1, o, tr['out_W'], tr['out_b'], tr['ln_g'], tr['ln_b'], tr['ff1_W'],
      tr['ff1_b'], tr['ff2_W'], tr['ff2_b'], tr['o_W'], tr['o_b'],
      tr['lnout_g'], tr['lnout_b'], c1['Wl'], c1['bl'], c1['Wr'], c1['br'])


def _ee_body(nheads, ea_ref, We_ref, *outs):
    i = pl.program_id(0)
    ea = ea_ref[...]
    easum_ref = outs[-1]

    @pl.when(i == 0)
    def _():
        easum_ref[...] = jnp.zeros_like(easum_ref)

    easum_ref[...] += ea.sum(0, keepdims=True)
    C = We_ref.shape[1] // nheads
    eef = ea @ We_ref[...]
    for h in range(nheads):
        outs[h][...] = eef[:, h * C:(h + 1) * C]


def _ee(edge_attr, We, nheads):
    C = We.shape[1] // nheads
    full = lambda s: pl.BlockSpec(s, lambda *a: (0,) * len(s))
    return pl.pallas_call(
        functools.partial(_ee_body, nheads),
        grid=(EB,),
        in_specs=[pl.BlockSpec((BE, DE), lambda i: (i, 0)),
                  full(We.shape)],
        out_specs=[pl.BlockSpec((BE, C), lambda i: (i, 0))] * nheads
        + [full((1, DE))],
        out_shape=[jax.ShapeDtypeStruct((E, C), jnp.float32)] * nheads
        + [jax.ShapeDtypeStruct((1, DE), jnp.float32)],
    )(edge_attr, We)


def _edges_jnp(xls, xrs, ees, src, dst, att, C, W):
    rows = []
    for h in range(4):
        u = xls[h][src] + xrs[h][dst] + ees[h]
        lk = jnp.where(u > 0, u, 0.2 * u)
        logit = (lk * att[h]).sum(-1)
        w = jnp.exp(logit)
        num = jax.ops.segment_sum(w[:, None] * xls[h][src], dst,
                                  num_segments=N)
        den = jax.ops.segment_sum(w, dst, num_segments=N)
        pad = jnp.zeros((N, W - C - 1), jnp.float32)
        rows.append(jnp.concatenate([num, den[:, None], pad], axis=-1))
    return jnp.stack(rows)


def _fin1_body(A_ref, easum_ref, We_ref, att_ref, bias_ref,
               bng_ref, bnb_ref, bnm_ref, bnv_ref,
               Wl2_ref, bl2_ref, Wr2_ref, br2_ref, *refs):
    xl_refs = refs[:4]
    xr_refs = refs[4:8]
    outs = refs[8:]
    c = (easum_ref[...] * (1.0 / E)) @ We_ref[...]
    att = att_ref[...]
    cols = []
    for h in range(4):
        xlh = xl_refs[h][...]
        xrh = xr_refs[h][...]
        u = xlh + xrh + c[:, h * 256:(h + 1) * 256]
        lk = jnp.where(u > 0, u, 0.2 * u)
        logit = (lk * att[h][None]).sum(-1, keepdims=True)
        w = jnp.exp(logit)
        Ah = A_ref[h]
        num = Ah[:, :256] + w * xlh
        den = Ah[:, 256:257] + w
        cols.append(num / den)
    x = jnp.concatenate(cols, axis=-1) + bias_ref[...]
    x = jnp.maximum(x, 0.0)
    x = (x - bnm_ref[...]) * jax.lax.rsqrt(bnv_ref[...] + 1e-5) \
        * bng_ref[...] + bnb_ref[...]
    xl2 = x @ Wl2_ref[...] + bl2_ref[...]
    xr2 = x @ Wr2_ref[...] + br2_ref[...]
    for h in range(4):
        outs[h][...] = xl2[:, h * 64:(h + 1) * 64]
        outs[4 + h][...] = xr2[:, h * 64:(h + 1) * 64]


def _fin1(A, easum, p, xls, xrs, c2p):
    full = lambda s: pl.BlockSpec(s, lambda i: (0,) * len(s))
    return pl.pallas_call(
        _fin1_body,
        grid=(NB,),
        in_specs=[pl.BlockSpec((4, BN, 272), lambda i: (0, i, 0)),
                  full((1, DE)), full((DE, 1024)), full((4, 256)),
                  full((1, 1024)), full((1, 1024)), full((1, 1024)),
                  full((1, 1024)), full((1, 1024)),
                  full((1024, 256)), full((1, 256)),
                  full((1024, 256)), full((1, 256))]
        + [pl.BlockSpec((BN, 256), lambda i: (i, 0))] * 8,
        out_specs=[pl.BlockSpec((BN, 64), lambda i: (i, 0))] * 8,
        out_shape=[jax.ShapeDtypeStruct((N, 64), jnp.float32)] * 8,
    )(A, easum, p['c1']['We'], p['c1']['att'], p['c1']['bias'],
      p['bn1']['g'], p['bn1']['b'], p['bn1']['m'], p['bn1']['v'],
      c2p['Wl'], c2p['bl'], c2p['Wr'], c2p['br'], *xls, *xrs)


def _fin2_body(A_ref, easum_ref, We_ref, att_ref, bias_ref,
               bng_ref, bnb_ref, bnm_ref, bnv_ref, batch_ref,
               fc1W_ref, fc1b_ref, fc2W_ref, fc2b_ref, *refs):
    xl_refs = refs[:4]
    xr_refs = refs[4:8]
    y_ref = refs[8]
    c = (easum_ref[...] * (1.0 / E)) @ We_ref[...]
    att = att_ref[...]
    cols = []
    for h in range(4):
        xlh = xl_refs[h][...]
        xrh = xr_refs[h][...]
        u = xlh + xrh + c[:, h * 64:(h + 1) * 64]
        lk = jnp.where(u > 0, u, 0.2 * u)
        logit = (lk * att[h][None]).sum(-1, keepdims=True)
        w = jnp.exp(logit)
        Ah = A_ref[h]
        num = Ah[:, :64] + w * xlh
        den = Ah[:, 64:65] + w
        cols.append(num / den)
    x = jnp.concatenate(cols, axis=-1) + bias_ref[...]
    x = jnp.maximum(x, 0.0)
    x = (x - bnm_ref[...]) * jax.lax.rsqrt(bnv_ref[...] + 1e-5) \
        * bng_ref[...] + bnb_ref[...]
    gid = lax.broadcasted_iota(jnp.int32, (G, N), 0)
    onehot = (batch_ref[...] == gid).astype(jnp.float32)
    pooled = onehot @ x
    y = jnp.maximum(pooled @ fc1W_ref[...] + fc1b_ref[...], 0.0)
    y_ref[...] = y @ fc2W_ref[...] + fc2b_ref[...]


def _fin2(A, easum, p, xls, xrs, batch):
    full = lambda s: pl.BlockSpec(s, lambda: (0,) * len(s))
    return pl.pallas_call(
        _fin2_body,
        in_specs=[full((4, N, 80)), full((1, DE)), full((DE, 256)),
                  full((4, 64)), full((1, 256)), full((1, 256)),
                  full((1, 256)), full((1, 256)), full((1, 256)),
                  full((1, N)), full((256, 64)), full((1, 64)),
                  full((64, 1)), full((1, 1))]
        + [full((N, 64))] * 8,
        out_specs=full((G, 1)),
        out_shape=jax.ShapeDtypeStruct((G, 1), jnp.float32),
    )(A, easum, p['c2']['We'], p['c2']['att'], p['c2']['bias'],
      p['bn2']['g'], p['bn2']['b'], p['bn2']['m'], p['bn2']['v'],
      batch.reshape(1, N), p['fc1_W'], p['fc1_b'], p['fc2_W'], p['fc2_b'],
      *xls, *xrs)


def kernel(x, edge_attr, params, edge_index, batch):
    p = params
    r2 = lambda a: a.reshape(1, -1)
    tr = dict(p['tr'])
    for k in ('in_b', 'ln_g', 'ln_b', 'qkv_b', 'out_b', 'ff1_b', 'ff2_b',
              'o_b', 'lnout_g', 'lnout_b'):
        tr[k] = r2(tr[k])
    c1 = dict(p['c1'])
    c2 = dict(p['c2'])
    for c in (c1, c2):
        c['bl'] = r2(c['bl'])
        c['br'] = r2(c['br'])
        c['bias'] = r2(c['bias'])
    pp = {'c1': c1, 'c2': c2,
          'bn1': {k: r2(v) for k, v in p['bn1'].items()},
          'bn2': {k: r2(v) for k, v in p['bn2'].items()},
          'fc1_W': p['fc1_W'], 'fc1_b': r2(p['fc1_b']),
          'fc2_W': p['fc2_W'], 'fc2_b': r2(p['fc2_b'])}
    src = edge_index[0]
    dst = edge_index[1]

    x1, qkv = _pre(x, tr)
    o = _attn(qkv)
    outs = _post(x1, o, tr, c1)
    xl1, xr1 = list(outs[:4]), list(outs[4:])
    ee1 = _ee(edge_attr, c1['We'], 4)
    ees1, easum = list(ee1[:4]), ee1[4]

    A1 = _edges_jnp(xl1, xr1, ees1, src, dst, c1['att'], 256, 272)

    outs2 = _fin1(A1, easum, pp, xl1, xr1, c2)
    xl2, xr2 = list(outs2[:4]), list(outs2[4:])
    ee2 = _ee(edge_attr, c2['We'], 4)
    ees2 = list(ee2[:4])

    A2 = _edges_jnp(xl2, xr2, ees2, src, dst, c2['att'], 64, 80)

    return _fin2(A2, easum, pp, xl2, xr2, batch)


# speedup vs baseline: 7.4922x; 2.3483x over previous
import functools
import math

import jax
import jax.numpy as jnp
from jax import lax
from jax.experimental import pallas as pl
from jax.experimental.pallas import tpu as pltpu
from jax.experimental.pallas import tpu_sc as plsc

N = 4096
E = 65536
DIN = 128
DE = 16
G = 32
NB = 8
BN = N // NB
EB = 64
BE = E // EB


def _ln(x, g, b):
    m = x.mean(-1, keepdims=True)
    v = ((x - m) ** 2).mean(-1, keepdims=True)
    return (x - m) * jax.lax.rsqrt(v + 1e-5) * g + b


def _pre_body(x_ref, inW_ref, inb_ref, lng_ref, lnb_ref, qkvW_ref, qkvb_ref,
              x1_ref, qkv_ref):
    x = x_ref[...]
    x1 = _ln(x @ inW_ref[...] + inb_ref[...], lng_ref[...], lnb_ref[...])
    x1_ref[...] = x1
    qkv_ref[...] = x1 @ qkvW_ref[...].T + qkvb_ref[...]


def _pre(x, tr):
    full = lambda s: pl.BlockSpec(s, lambda i: (0,) * len(s))
    return pl.pallas_call(
        _pre_body,
        grid=(NB,),
        in_specs=[pl.BlockSpec((BN, DIN), lambda i: (i, 0)),
                  full((DIN, 128)), full((1, 128)), full((1, 128)),
                  full((1, 128)), full((384, 128)), full((1, 384))],
        out_specs=[pl.BlockSpec((BN, 128), lambda i: (i, 0)),
                   pl.BlockSpec((BN, 384), lambda i: (i, 0))],
        out_shape=[jax.ShapeDtypeStruct((N, 128), jnp.float32),
                   jax.ShapeDtypeStruct((N, 384), jnp.float32)],
    )(x, tr['in_W'], tr['in_b'], tr['ln_g'], tr['ln_b'], tr['qkv_W'],
      tr['qkv_b'])


def _attn_body(q_ref, kv_ref, o_ref):
    qblk = q_ref[...]
    kv = kv_ref[...]
    outs = []
    for h in range(4):
        q = qblk[:, 32 * h:32 * h + 32]
        k = kv[:, 128 + 32 * h:128 + 32 * h + 32]
        v = kv[:, 256 + 32 * h:256 + 32 * h + 32]
        s = (q @ k.T) * (1.0 / math.sqrt(32.0))
        m = s.max(-1, keepdims=True)
        p = jnp.exp(s - m)
        p = p / p.sum(-1, keepdims=True)
        outs.append(p @ v)
    o_ref[...] = jnp.concatenate(outs, axis=-1)


def _attn(qkv):
    return pl.pallas_call(
        _attn_body,
        grid=(NB,),
        in_specs=[pl.BlockSpec((BN, 384), lambda i: (i, 0)),
                  pl.BlockSpec((N, 384), lambda i: (0, 0))],
        out_specs=pl.BlockSpec((BN, 128), lambda i: (i, 0)),
        out_shape=jax.ShapeDtypeStruct((N, 128), jnp.float32),
    )(qkv, qkv)


def _post_body(x1_ref, o_ref, outW_ref, outb_ref, lng_ref, lnb_ref,
               ff1W_ref, ff1b_ref, ff2W_ref, ff2b_ref, oW_ref, ob_ref,
               lnog_ref, lnob_ref, Wl_ref, bl_ref, Wr_ref, br_ref, *outs):
    x = x1_ref[...] + (o_ref[...] @ outW_ref[...].T + outb_ref[...])
    x = _ln(x, lng_ref[...], lnb_ref[...])
    ff = jnp.maximum(x @ ff1W_ref[...] + ff1b_ref[...], 0.0) @ ff2W_ref[...] \
        + ff2b_ref[...]
    x = _ln(x + ff, lng_ref[...], lnb_ref[...])
    x = x @ oW_ref[...] + ob_ref[...]
    x = _ln(x, lnog_ref[...], lnob_ref[...])
    xl = x @ Wl_ref[...] + bl_ref[...]
    xr = x @ Wr_ref[...] + br_ref[...]
    for h in range(4):
        outs[h][...] = xl[:, h * 256:(h + 1) * 256]
        outs[4 + h][...] = xr[:, h * 256:(h + 1) * 256]


def _post(x1, o, tr, c1):
    full = lambda s: pl.BlockSpec(s, lambda i: (0,) * len(s))
    return pl.pallas_call(
        _post_body,
        grid=(NB,),
        in_specs=[pl.BlockSpec((BN, 128), lambda i: (i, 0)),
                  pl.BlockSpec((BN, 128), lambda i: (i, 0)),
                  full((128, 128)), full((1, 128)), full((1, 128)),
                  full((1, 128)), full((128, 512)), full((1, 512)),
                  full((512, 128)), full((1, 128)), full((128, 64)),
                  full((1, 64)), full((1, 64)), full((1, 64)),
                  full((64, 1024)), full((1, 1024)),
                  full((64, 1024)), full((1, 1024))],
        out_specs=[pl.BlockSpec((BN, 256), lambda i: (i, 0))] * 8,
        out_shape=[jax.ShapeDtypeStruct((N, 256), jnp.float32)] * 8,
    )(x1, o, tr['out_W'], tr['out_b'], tr['ln_g'], tr['ln_b'], tr['ff1_W'],
      tr['ff1_b'], tr['ff2_W'], tr['ff2_b'], tr['o_W'], tr['o_b'],
      tr['lnout_g'], tr['lnout_b'], c1['Wl'], c1['bl'], c1['Wr'], c1['br'])


def _ee_body(nheads, ea_ref, We_ref, *outs):
    i = pl.program_id(0)
    ea = ea_ref[...]
    easum_ref = outs[-1]

    @pl.when(i == 0)
    def _():
        easum_ref[...] = jnp.zeros_like(easum_ref)

    easum_ref[...] += ea.sum(0, keepdims=True)
    C = We_ref.shape[1] // nheads
    TW = outs[0].shape[-1]
    eef = ea @ We_ref[...]
    for h in range(nheads):
        blk = eef[:, h * C:(h + 1) * C]
        if TW > C:
            blk = jnp.concatenate(
                [blk, jnp.zeros((blk.shape[0], TW - C), jnp.float32)], -1)
        outs[h][...] = blk


def _ee(edge_attr, We, nheads, TW):
    full = lambda s: pl.BlockSpec(s, lambda *a: (0,) * len(s))
    return pl.pallas_call(
        functools.partial(_ee_body, nheads),
        grid=(EB,),
        in_specs=[pl.BlockSpec((BE, DE), lambda i: (i, 0)),
                  full(We.shape)],
        out_specs=[pl.BlockSpec((BE, TW), lambda i: (i, 0))] * nheads
        + [full((1, DE))],
        out_shape=[jax.ShapeDtypeStruct((E, TW), jnp.float32)] * nheads
        + [jax.ShapeDtypeStruct((1, DE), jnp.float32)],
    )(edge_attr, We)


_GDN = lax.GatherDimensionNumbers(offset_dims=(), collapsed_slice_dims=(0,),
                                  start_index_map=(0,))


def _lane_shuffle(x, idx):
    return lax.gather(x, idx[:, None], _GDN, (1,),
                      mode=lax.GatherScatterMode.PROMISE_IN_BOUNDS)


def _make_gat_edges_sc(C, nhalves, CH, nG):
    SL = C // 128
    Hc = C // nhalves
    EPT = E // 16
    RPT = N // 16
    PASSES = nG // 2
    mesh = plsc.VectorSubcoreMesh(core_axis_name="c", subcore_axis_name="s")

    def cd(col):
        return (col // 128, pl.ds(col % 128, 16))

    def body(*refs):
        src_hbm, dst_hbm = refs[0], refs[1]
        xls = refs[2:2 + nG]
        xrs = refs[2 + nG:2 + 2 * nG]
        ees = refs[2 + 2 * nG:2 + 3 * nG]
        att_hbm = refs[2 + 3 * nG]
        A_hbm, den_hbm = refs[3 + 3 * nG], refs[4 + 3 * nG]
        (srcv, dstv, xlv, xrv, eev, denrow, attv, sem) = \
            refs[5 + 3 * nG:5 + 3 * nG + 8]
        acc_sh, den_sh = refs[5 + 3 * nG + 8], refs[5 + 3 * nG + 9]
        cid = lax.axis_index("c")
        sid = lax.axis_index("s")
        lane = lax.iota(jnp.int32, 16)
        z16 = jnp.zeros((16,), jnp.float32)

        def zdr(r, _):
            for j in range(8):
                denrow[r, 0, pl.ds(16 * j, 16)] = z16
            return 0
        lax.fori_loop(0, CH, zdr, 0)

        for g in range(nG):
            core = g // PASSES
            first = (g % PASSES) == 0
            last = (g % PASSES) == PASSES - 1

            @pl.when(cid == core)
            def _(g=g, core=core, first=first, last=last):
                pltpu.sync_copy(att_hbm.at[g], attv)

                def zx(r, _):
                    for j in range(C // 16):
                        s, d = cd(16 * j)
                        xlv[r, s, d] = z16
                    return 0
                lax.fori_loop(0, CH, zx, 0)
                for z in range(RPT // CH):
                    rs = sid * RPT + z * CH
                    pltpu.sync_copy(xlv, acc_sh.at[pl.ds(rs, CH)])
                    if first:
                        pltpu.sync_copy(denrow, den_sh.at[pl.ds(rs, CH)])
                plsc.subcore_barrier()

                def chunk(ci, _):
                    base = sid * EPT + ci * CH
                    pltpu.sync_copy(src_hbm.at[pl.ds(base, CH)], srcv)
                    pltpu.sync_copy(dst_hbm.at[pl.ds(base, CH)], dstv)
                    pltpu.async_copy(xls[g].at[srcv], xlv, sem).wait()
                    pltpu.async_copy(xrs[g].at[dstv], xrv, sem).wait()
                    pltpu.sync_copy(ees[g].at[pl.ds(base, CH)], eev)

                    def edge(e, _):
                        ws = []
                        for k in range(nhalves):
                            acc = z16
                            for j in range(k * Hc // 16,
                                           (k + 1) * Hc // 16):
                                s, d = cd(16 * j)
                                u = (xlv[e, s, d] + xrv[e, s, d]
                                     + eev[e, s, d])
                                lk = jnp.where(u > 0.0, u, 0.2 * u)
                                acc = acc + lk * attv[pl.ds(16 * j, 16)]
                            for s in (8, 4, 2, 1):
                                acc = acc + _lane_shuffle(
                                    acc, jnp.bitwise_xor(lane, s))
                            ws.append(jnp.exp(acc))
                        dv = z16
                        for k in range(nhalves):
                            dv = dv + jnp.where(
                                lane == g * nhalves + k, ws[k], 0.0)
                        denrow[e, 0, pl.ds(0, 16)] = dv
                        for k in range(nhalves):
                            for j in range(k * Hc // 16,
                                           (k + 1) * Hc // 16):
                                s, d = cd(16 * j)
                                xlv[e, s, d] = xlv[e, s, d] * ws[k]
                        return 0
                    lax.fori_loop(0, CH, edge, 0)
                    pltpu.sync_copy(xlv, acc_sh.at[dstv], add=True)
                    pltpu.sync_copy(denrow, den_sh.at[dstv], add=True)
                    return 0
                lax.fori_loop(0, EPT // CH, chunk, 0)
                plsc.subcore_barrier()
                for z in range(RPT // CH):
                    rs = sid * RPT + z * CH
                    pltpu.sync_copy(acc_sh.at[pl.ds(rs, CH)], xlv)
                    pltpu.sync_copy(xlv, A_hbm.at[g, pl.ds(rs, CH)])
                    if last:
                        pltpu.sync_copy(den_sh.at[pl.ds(rs, CH)], denrow)
                        pltpu.sync_copy(denrow,
                                        den_hbm.at[core, pl.ds(rs, CH)])
                plsc.subcore_barrier()

    return pl.kernel(
        body,
        out_type=[jax.ShapeDtypeStruct((nG, N, SL, 128), jnp.float32),
                  jax.ShapeDtypeStruct((2, N, 1, 128), jnp.float32)],
        mesh=mesh,
        scratch_types=[
            pltpu.VMEM((CH,), jnp.int32),
            pltpu.VMEM((CH,), jnp.int32),
            pltpu.VMEM((CH, SL, 128), jnp.float32),
            pltpu.VMEM((CH, SL, 128), jnp.float32),
            pltpu.VMEM((CH, SL, 128), jnp.float32),
            pltpu.VMEM((CH, 1, 128), jnp.float32),
            pltpu.VMEM((C,), jnp.float32),
            pltpu.SemaphoreType.DMA,
            pltpu.VMEM_SHARED((N, SL, 128), jnp.float32),
            pltpu.VMEM_SHARED((N, 1, 128), jnp.float32),
        ],
    )


def _fin1_body(A_ref, den_ref, easum_ref, We_ref, att_ref, bias_ref,
               bng_ref, bnb_ref, bnm_ref, bnv_ref,
               Wl2_ref, bl2_ref, Wr2_ref, br2_ref, *refs):
    xl_refs = refs[:4]
    xr_refs = refs[4:8]
    outs = refs[8:]
    c = (easum_ref[...] * (1.0 / E)) @ We_ref[...]
    att = att_ref[...]
    dena = den_ref[...]
    cols = []
    for h in range(4):
        xlh = xl_refs[h][...]
        xrh = xr_refs[h][...]
        u = xlh + xrh + c[:, h * 256:(h + 1) * 256]
        lk = jnp.where(u > 0, u, 0.2 * u)
        logit = (lk * att[h][None]).sum(-1, keepdims=True)
        w = jnp.exp(logit)
        num = A_ref[h] + w * xlh
        den = dena[h // 2][:, h:h + 1] + w
        cols.append(num / den)
    x = jnp.concatenate(cols, axis=-1) + bias_ref[...]
    x = jnp.maximum(x, 0.0)
    x = (x - bnm_ref[...]) * jax.lax.rsqrt(bnv_ref[...] + 1e-5) \
        * bng_ref[...] + bnb_ref[...]
    xl2 = x @ Wl2_ref[...] + bl2_ref[...]
    xr2 = x @ Wr2_ref[...] + br2_ref[...]
    for k in range(2):
        outs[k][...] = xl2[:, k * 128:(k + 1) * 128]
        outs[2 + k][...] = xr2[:, k * 128:(k + 1) * 128]


def _fin1(A, den, easum, p, xls, xrs, c2p):
    full = lambda s: pl.BlockSpec(s, lambda i: (0,) * len(s))
    return pl.pallas_call(
        _fin1_body,
        grid=(NB,),
        in_specs=[pl.BlockSpec((4, BN, 256), lambda i: (0, i, 0)),
                  pl.BlockSpec((2, BN, 128), lambda i: (0, i, 0)),
                  full((1, DE)), full((DE, 1024)), full((4, 256)),
                  full((1, 1024)), full((1, 1024)), full((1, 1024)),
                  full((1, 1024)), full((1, 1024)),
                  full((1024, 256)), full((1, 256)),
                  full((1024, 256)), full((1, 256))]
        + [pl.BlockSpec((BN, 256), lambda i: (i, 0))] * 8,
        out_specs=[pl.BlockSpec((BN, 128), lambda i: (i, 0))] * 4,
        out_shape=[jax.ShapeDtypeStruct((N, 128), jnp.float32)] * 4,
    )(A, den, easum, p['c1']['We'], p['c1']['att'], p['c1']['bias'],
      p['bn1']['g'], p['bn1']['b'], p['bn1']['m'], p['bn1']['v'],
      c2p['Wl'], c2p['bl'], c2p['Wr'], c2p['br'], *xls, *xrs)


def _fin2_body(A_ref, den_ref, easum_ref, We_ref, att_ref, bias_ref,
               bng_ref, bnb_ref, bnm_ref, bnv_ref, batch_ref,
               fc1W_ref, fc1b_ref, fc2W_ref, fc2b_ref, *refs):
    xl_refs = refs[:2]
    xr_refs = refs[2:4]
    y_ref = refs[4]
    c = (easum_ref[...] * (1.0 / E)) @ We_ref[...]
    att = att_ref[...]
    dena = den_ref[...]
    cols = []
    for h in range(4):
        q = (h % 2) * 64
        xlh = xl_refs[h // 2][...][:, q:q + 64]
        xrh = xr_refs[h // 2][...][:, q:q + 64]
        u = xlh + xrh + c[:, h * 64:(h + 1) * 64]
        lk = jnp.where(u > 0, u, 0.2 * u)
        logit = (lk * att[h][None]).sum(-1, keepdims=True)
        w = jnp.exp(logit)
        num = A_ref[h // 2][:, q:q + 64] + w * xlh
        den = dena[h // 2][:, h:h + 1] + w
        cols.append(num / den)
    x = jnp.concatenate(cols, axis=-1) + bias_ref[...]
    x = jnp.maximum(x, 0.0)
    x = (x - bnm_ref[...]) * jax.lax.rsqrt(bnv_ref[...] + 1e-5) \
        * bng_ref[...] + bnb_ref[...]
    gid = lax.broadcasted_iota(jnp.int32, (G, N), 0)
    onehot = (batch_ref[...] == gid).astype(jnp.float32)
    pooled = onehot @ x
    y = jnp.maximum(pooled @ fc1W_ref[...] + fc1b_ref[...], 0.0)
    y_ref[...] = y @ fc2W_ref[...] + fc2b_ref[...]


def _fin2(A, den, easum, p, xls, xrs, batch):
    full = lambda s: pl.BlockSpec(s, lambda: (0,) * len(s))
    return pl.pallas_call(
        _fin2_body,
        in_specs=[full((2, N, 128)), full((2, N, 128)),
                  full((1, DE)), full((DE, 256)),
                  full((4, 64)), full((1, 256)), full((1, 256)),
                  full((1, 256)), full((1, 256)), full((1, 256)),
                  full((1, N)), full((256, 64)), full((1, 64)),
                  full((64, 1)), full((1, 1))]
        + [full((N, 128))] * 4,
        out_specs=full((G, 1)),
        out_shape=jax.ShapeDtypeStruct((G, 1), jnp.float32),
    )(A, den, easum, p['c2']['We'], p['c2']['att'], p['c2']['bias'],
      p['bn2']['g'], p['bn2']['b'], p['bn2']['m'], p['bn2']['v'],
      batch.reshape(1, N), p['fc1_W'], p['fc1_b'], p['fc2_W'], p['fc2_b'],
      *xls, *xrs)


def kernel(x, edge_attr, params, edge_index, batch):
    p = params
    r2 = lambda a: a.reshape(1, -1)
    tr = dict(p['tr'])
    for k in ('in_b', 'ln_g', 'ln_b', 'qkv_b', 'out_b', 'ff1_b', 'ff2_b',
              'o_b', 'lnout_g', 'lnout_b'):
        tr[k] = r2(tr[k])
    c1 = dict(p['c1'])
    c2 = dict(p['c2'])
    for c in (c1, c2):
        c['bl'] = r2(c['bl'])
        c['br'] = r2(c['br'])
        c['bias'] = r2(c['bias'])
    pp = {'c1': c1, 'c2': c2,
          'bn1': {k: r2(v) for k, v in p['bn1'].items()},
          'bn2': {k: r2(v) for k, v in p['bn2'].items()},
          'fc1_W': p['fc1_W'], 'fc1_b': r2(p['fc1_b']),
          'fc2_W': p['fc2_W'], 'fc2_b': r2(p['fc2_b'])}
    src = edge_index[0]
    dst = edge_index[1]

    x1, qkv = _pre(x, tr)
    o = _attn(qkv)
    outs = _post(x1, o, tr, c1)
    xl1, xr1 = list(outs[:4]), list(outs[4:])
    ee1 = _ee(edge_attr, c1['We'], 4, 256)
    ees1, easum = list(ee1[:4]), ee1[4]

    r3 = lambda a: a.reshape(a.shape[0], -1, 128)
    A1, den1 = _make_gat_edges_sc(256, 1, 32, 4)(
        src, dst, *[r3(a) for a in xl1], *[r3(a) for a in xr1],
        *[r3(a) for a in ees1], c1['att'])
    A1 = A1.reshape(4, N, 256)
    den1 = den1.reshape(2, N, 128)

    outs2 = _fin1(A1, den1, easum, pp, xl1, xr1, c2)
    xl2, xr2 = list(outs2[:2]), list(outs2[2:])
    ee2 = _ee(edge_attr, c2['We'], 2, 128)
    ees2 = list(ee2[:2])

    A2, den2 = _make_gat_edges_sc(128, 2, 64, 2)(
        src, dst, *[r3(a) for a in xl2], *[r3(a) for a in xr2],
        *[r3(a) for a in ees2], c2['att'].reshape(2, 128))
    A2 = A2.reshape(2, N, 128)
    den2 = den2.reshape(2, N, 128)

    return _fin2(A2, den2, easum, pp, xl2, xr2, batch)
